# baseline (speedup 1.0000x reference)
import functools

import jax
import jax.numpy as jnp
from jax import lax
from jax.experimental import pallas as pl
from jax.experimental.pallas import tpu as pltpu
from jax.experimental.pallas import tpu_sc as plsc

_N = 10000
_NP = 10240
_H1 = 16

_NC = 2
_NS = 16
_NW = _NC * _NS
_G = 128
_RPT = _NP // _NS


def _tc_linear1(x, W1):
    n = x.shape[0]

    def body(x_ref, wt_ref, o_ref):
        o_ref[pl.ds(0, n), :] = jax.lax.dot_general(
            x_ref[...], wt_ref[...], (((1,), (1,)), ((), ())),
            preferred_element_type=jnp.float32)
        o_ref[pl.ds(n, _NP - n), :] = jnp.zeros((_NP - n, _H1), jnp.float32)

    return pl.pallas_call(
        body,
        out_shape=jax.ShapeDtypeStruct((_NP, _H1), jnp.float32),
    )(x, W1)


def _tc_mid(p, b1r, W2p):
    def body(p_ref, b_ref, w_ref, o_ref):
        w = w_ref[...]
        b = b_ref[...]
        for i in range(8):
            sl = pl.ds(i * _H1, _H1)
            h = jnp.maximum(p_ref[0, :, sl] + p_ref[1, :, sl] + b, 0.0)
            o_ref[:, sl] = jnp.dot(h, w, preferred_element_type=jnp.float32)

    return pl.pallas_call(
        body,
        out_shape=jax.ShapeDtypeStruct((p.shape[1], 128), jnp.float32),
    )(p, b1r, W2p)


def _tc_final(p, b2r):
    def body(p_ref, b_ref, o_ref):
        b = b_ref[...]
        for i in range(8):
            sl = pl.ds(i * _H1, _H1)
            o_ref[:, sl] = p_ref[0, :, sl] + p_ref[1, :, sl] + b

    return pl.pallas_call(
        body,
        out_shape=jax.ShapeDtypeStruct((p.shape[1], 128), jnp.float32),
    )(p, b2r)


def _make_agg(num_groups):
    mesh = plsc.VectorSubcoreMesh(core_axis_name="c", subcore_axis_name="s")
    base = num_groups // _NW
    extra = num_groups % _NW

    @functools.partial(
        pl.kernel,
        out_type=jax.ShapeDtypeStruct((_NC, _NP, _H1), jnp.float32),
        mesh=mesh,
        compiler_params=pltpu.CompilerParams(use_tc_tiling_on_sc=False),
        scratch_types=[
            pltpu.VMEM_SHARED((_NP, _H1), jnp.float32),
            pltpu.VMEM_SHARED((_NP, _H1), jnp.float32),
            pltpu.VMEM((base + 1, 2, _G), jnp.int32),
            pltpu.VMEM((base + 1, _G), jnp.float32),
            pltpu.VMEM((2, 8, _G, _H1), jnp.float32),
            pltpu.VMEM((2, 8, _G, _H1), jnp.float32),
            pltpu.VMEM((_RPT, _H1), jnp.float32),
            pltpu.SemaphoreType.DMA((2, 8)),
            pltpu.SemaphoreType.DMA((2,)),
            pltpu.SemaphoreType.DMA,
            pltpu.SemaphoreType.DMA,
            pltpu.SemaphoreType.DMA,
        ],
    )
    def agg(table_hbm, ei_hbm, ew_hbm, out_hbm,
            table_sh, acc_sh, ei_v, w_v, rows_v, srows_v, zbuf,
            gsem, ssem, esem, tsem, stsem):
        c = lax.axis_index("c")
        s = lax.axis_index("s")
        wid = s * _NC + c
        ng = base + jnp.where(wid < extra, 1, 0)
        g0 = wid * base + jnp.minimum(wid, extra)
        r0 = s * _RPT

        st1 = pltpu.async_copy(table_hbm.at[pl.ds(r0, _RPT)],
                               table_sh.at[pl.ds(r0, _RPT)], stsem)
        e1 = pltpu.async_copy(ei_hbm.at[pl.ds(g0, base)],
                              ei_v.at[pl.ds(0, base)], esem)
        e2 = pltpu.async_copy(ew_hbm.at[pl.ds(g0, base)],
                              w_v.at[pl.ds(0, base)], esem)

        @pl.when(wid < extra)
        def _fetch_tail():
            pltpu.async_copy(ei_hbm.at[pl.ds(g0 + base, 1)],
                             ei_v.at[pl.ds(base, 1)], tsem)
            pltpu.async_copy(ew_hbm.at[pl.ds(g0 + base, 1)],
                             w_v.at[pl.ds(base, 1)], tsem)

        def zrow(j, cc):
            zbuf[j, :] = jnp.zeros((_H1,), jnp.float32)
            return cc

        lax.fori_loop(0, _RPT, zrow, 0)
        pltpu.sync_copy(zbuf, acc_sh.at[pl.ds(r0, _RPT)])
        st1.wait()
        plsc.subcore_barrier()
        e1.wait()
        e2.wait()

        @pl.when(wid < extra)
        def _wait_tail():
            pltpu.make_async_copy(ei_hbm.at[pl.ds(g0 + base, 1)],
                                  ei_v.at[pl.ds(base, 1)], tsem).wait()
            pltpu.make_async_copy(ew_hbm.at[pl.ds(g0 + base, 1)],
                                  w_v.at[pl.ds(base, 1)], tsem).wait()

        nch = (base + 1 + 7) // 8

        def gather_args(g, p, a):
            return (table_sh.at[ei_v.at[g, 1]], rows_v.at[p, a],
                    gsem.at[p, a])

        def scatter_args(g, p, a):
            return (srows_v.at[p, a], acc_sh.at[ei_v.at[g, 0]], ssem.at[p])

        def issue_chunk_gathers(i, p):
            for a in range(8):
                g = i * 8 + a

                @pl.when(jnp.logical_and(i < nch, g < ng))
                def _issue(g=g, a=a):
                    pltpu.async_copy(*gather_args(g, p, a))

        def drain_chunk_scatters(i, p):
            for a in range(8):
                g = i * 8 + a

                @pl.when(jnp.logical_and(i >= 0, g < ng))
                def _drain(g=g, a=a):
                    pltpu.make_async_copy(*scatter_args(g, p, a)).wait()

        issue_chunk_gathers(jnp.int32(0), 0)

        def chunk_body(i, carry):
            p = lax.rem(i, 2)
            drain_chunk_scatters(i - 2, p)
            issue_chunk_gathers(i + 1, 1 - p)
            for a in range(8):
                g = i * 8 + a

                @pl.when(g < ng)
                def _process(g=g, a=a):
                    pltpu.make_async_copy(*gather_args(g, p, a)).wait()

                    def scale(b, cc):
                        j0 = b * 16
                        wvec = w_v[g, pl.ds(j0, 16)]
                        for r in range(16):
                            srows_v[p, a, j0 + r, :] = (
                                rows_v[p, a, j0 + r, :] * wvec[r])
                        return cc

                    lax.fori_loop(0, _G // 16, scale, 0)
                    pltpu.async_copy(*scatter_args(g, p, a), add=True)

            return carry

        lax.fori_loop(0, nch, chunk_body, 0)
        for i in (nch - 2, nch - 1):
            drain_chunk_scatters(jnp.int32(i), i % 2)
        plsc.subcore_barrier()
        pltpu.sync_copy(acc_sh.at[pl.ds(r0, _RPT)], out_hbm.at[c, pl.ds(r0, _RPT)])

    return agg


def kernel(x, edge_index, edge_weight, W1, b1, W2, b2):
    E = edge_weight.shape[0]
    H2 = W2.shape[1]
    num_groups = E // _G
    assert num_groups * _G == E

    ei_t = edge_index.reshape(2, num_groups, _G).transpose(1, 0, 2)
    ew_r = edge_weight.reshape(num_groups, _G)

    agg = _make_agg(num_groups)

    xw = _tc_linear1(x, W1.T)
    p1 = agg(xw, ei_t, ew_r)

    W2p = jnp.pad(W2, ((0, 0), (0, _H1 - H2)))
    hw128 = _tc_mid(p1.reshape(_NC, _NP // 8, 128), b1.reshape(1, _H1), W2p)
    p2 = agg(hw128.reshape(_NP, _H1), ei_t, ew_r)

    b2r = jnp.pad(b2, (0, _H1 - H2)).reshape(1, _H1)
    out128 = _tc_final(p2.reshape(_NC, _NP // 8, 128), b2r)
    return out128.reshape(_NP, _H1)[:_N, :H2]


# speedup vs baseline: 1.8509x; 1.8509x over previous
import functools

import jax
import jax.numpy as jnp
from jax import lax
from jax.experimental import pallas as pl
from jax.experimental.pallas import tpu as pltpu
from jax.experimental.pallas import tpu_sc as plsc

_N = 10000
_NP = 10240
_H1 = 16

_NC = 2
_NS = 16
_NW = _NC * _NS
_G = 128
_RPT = _NP // _NS


def _tc_linear1(x, W1):
    n = x.shape[0]

    def body(x_ref, wt_ref, o_ref):
        o_ref[pl.ds(0, n), :] = jax.lax.dot_general(
            x_ref[...], wt_ref[...], (((1,), (1,)), ((), ())),
            preferred_element_type=jnp.float32)
        o_ref[pl.ds(n, _NP - n), :] = jnp.zeros((_NP - n, _H1), jnp.float32)

    return pl.pallas_call(
        body,
        out_shape=jax.ShapeDtypeStruct((_NP, _H1), jnp.float32),
    )(x, W1)


def _tc_mid(p, b1r, W2p):
    def body(p_ref, b_ref, w_ref, o_ref):
        w = w_ref[...]
        b = b_ref[...]
        for i in range(8):
            sl = pl.ds(i * _H1, _H1)
            h = jnp.maximum(p_ref[0, :, sl] + p_ref[1, :, sl] + b, 0.0)
            o_ref[:, sl] = jnp.dot(h, w, preferred_element_type=jnp.float32)

    return pl.pallas_call(
        body,
        out_shape=jax.ShapeDtypeStruct((p.shape[1], 128), jnp.float32),
    )(p, b1r, W2p)


def _tc_final(p, b2r):
    def body(p_ref, b_ref, o_ref):
        b = b_ref[...]
        for i in range(8):
            sl = pl.ds(i * _H1, _H1)
            o_ref[:, sl] = p_ref[0, :, sl] + p_ref[1, :, sl] + b

    return pl.pallas_call(
        body,
        out_shape=jax.ShapeDtypeStruct((p.shape[1], 128), jnp.float32),
    )(p, b2r)


def _make_agg(num_groups):
    mesh = plsc.VectorSubcoreMesh(core_axis_name="c", subcore_axis_name="s")
    base = num_groups // _NW
    extra = num_groups % _NW

    @functools.partial(
        pl.kernel,
        out_type=jax.ShapeDtypeStruct((_NC, _NP, _H1), jnp.float32),
        mesh=mesh,
        compiler_params=pltpu.CompilerParams(use_tc_tiling_on_sc=False),
        scratch_types=[
            pltpu.VMEM_SHARED((_NP, _H1), jnp.float32),
            pltpu.VMEM_SHARED((_NP, _H1), jnp.float32),
            pltpu.VMEM((base + 1, 2, _G), jnp.int32),
            pltpu.VMEM((base + 1, _G), jnp.float32),
            pltpu.VMEM((2, 8, _G, _H1), jnp.float32),
            pltpu.VMEM((2, 8, _G, _H1), jnp.float32),
            pltpu.VMEM((_RPT, _H1), jnp.float32),
            pltpu.SemaphoreType.DMA((2, 8)),
            pltpu.SemaphoreType.DMA((2,)),
            pltpu.SemaphoreType.DMA,
            pltpu.SemaphoreType.DMA,
            pltpu.SemaphoreType.DMA,
        ],
    )
    def agg(table_hbm, ei_hbm, ew_hbm, out_hbm,
            table_sh, acc_sh, ei_v, w_v, rows_v, srows_v, zbuf,
            gsem, ssem, esem, tsem, stsem):
        c = lax.axis_index("c")
        s = lax.axis_index("s")
        wid = s * _NC + c
        ng = base + jnp.where(wid < extra, 1, 0)
        g0 = wid * base + jnp.minimum(wid, extra)
        r0 = s * _RPT

        st1 = pltpu.async_copy(table_hbm.at[pl.ds(r0, _RPT)],
                               table_sh.at[pl.ds(r0, _RPT)], stsem)
        e1 = pltpu.async_copy(ei_hbm.at[pl.ds(g0, base)],
                              ei_v.at[pl.ds(0, base)], esem)
        e2 = pltpu.async_copy(ew_hbm.at[pl.ds(g0, base)],
                              w_v.at[pl.ds(0, base)], esem)

        @pl.when(wid < extra)
        def _fetch_tail():
            pltpu.async_copy(ei_hbm.at[pl.ds(g0 + base, 1)],
                             ei_v.at[pl.ds(base, 1)], tsem)
            pltpu.async_copy(ew_hbm.at[pl.ds(g0 + base, 1)],
                             w_v.at[pl.ds(base, 1)], tsem)

        def zrow(j, cc):
            zbuf[j, :] = jnp.zeros((_H1,), jnp.float32)
            return cc

        lax.fori_loop(0, _RPT, zrow, 0)
        pltpu.sync_copy(zbuf, acc_sh.at[pl.ds(r0, _RPT)])
        st1.wait()
        plsc.subcore_barrier()
        e1.wait()
        e2.wait()

        @pl.when(wid < extra)
        def _wait_tail():
            pltpu.make_async_copy(ei_hbm.at[pl.ds(g0 + base, 1)],
                                  ei_v.at[pl.ds(base, 1)], tsem).wait()
            pltpu.make_async_copy(ew_hbm.at[pl.ds(g0 + base, 1)],
                                  w_v.at[pl.ds(base, 1)], tsem).wait()

        nch = (base + 1 + 7) // 8

        def gather_args(g, p, a):
            return (table_sh.at[ei_v.at[g, 1]], rows_v.at[p, a],
                    gsem.at[p, a])

        def scatter_args(g, p, a):
            return (srows_v.at[p, a], acc_sh.at[ei_v.at[g, 0]], ssem.at[p])

        def issue_chunk_gathers(i, p):
            for a in range(8):
                g = i * 8 + a

                @pl.when(jnp.logical_and(i < nch, g < ng))
                def _issue(g=g, a=a):
                    pltpu.async_copy(*gather_args(g, p, a))

        def drain_chunk_scatters(i, p):
            for a in range(8):
                g = i * 8 + a

                @pl.when(jnp.logical_and(i >= 0, g < ng))
                def _drain(g=g, a=a):
                    pltpu.make_async_copy(*scatter_args(g, p, a)).wait()

        issue_chunk_gathers(jnp.int32(0), 0)
        assert nch % 2 == 0

        def process_chunk(i, p):
            for a in range(8):
                g = i * 8 + a

                @pl.when(g < ng)
                def _process(g=g, a=a):
                    pltpu.make_async_copy(*gather_args(g, p, a)).wait()

                    @plsc.parallel_loop(0, _G // 16)
                    def _scale(b):
                        j0 = b * 16
                        wvec = w_v[g, pl.ds(j0, 16)]
                        for r in range(16):
                            srows_v[p, a, j0 + r, :] = (
                                rows_v[p, a, j0 + r, :] * wvec[r])

                    pltpu.async_copy(*scatter_args(g, p, a), add=True)

        def pair_body(i2, carry):
            for k in range(2):
                i = i2 * 2 + k
                drain_chunk_scatters(i - 2, k)
                issue_chunk_gathers(i + 1, 1 - k)
                process_chunk(i, k)
            return carry

        lax.fori_loop(0, nch // 2, pair_body, 0)
        for i in (nch - 2, nch - 1):
            drain_chunk_scatters(jnp.int32(i), i % 2)
        plsc.subcore_barrier()
        pltpu.sync_copy(acc_sh.at[pl.ds(r0, _RPT)], out_hbm.at[c, pl.ds(r0, _RPT)])

    return agg


def kernel(x, edge_index, edge_weight, W1, b1, W2, b2):
    E = edge_weight.shape[0]
    H2 = W2.shape[1]
    num_groups = E // _G
    assert num_groups * _G == E

    ei_t = edge_index.reshape(2, num_groups, _G).transpose(1, 0, 2)
    ew_r = edge_weight.reshape(num_groups, _G)

    agg = _make_agg(num_groups)

    xw = _tc_linear1(x, W1.T)
    p1 = agg(xw, ei_t, ew_r)

    W2p = jnp.pad(W2, ((0, 0), (0, _H1 - H2)))
    hw128 = _tc_mid(p1.reshape(_NC, _NP // 8, 128), b1.reshape(1, _H1), W2p)
    p2 = agg(hw128.reshape(_NP, _H1), ei_t, ew_r)

    b2r = jnp.pad(b2, (0, _H1 - H2)).reshape(1, _H1)
    out128 = _tc_final(p2.reshape(_NC, _NP // 8, 128), b2r)
    return out128.reshape(_NP, _H1)[:_N, :H2]


# speedup vs baseline: 2.0264x; 1.0948x over previous
import functools

import jax
import jax.numpy as jnp
from jax import lax
from jax.experimental import pallas as pl
from jax.experimental.pallas import tpu as pltpu
from jax.experimental.pallas import tpu_sc as plsc

_N = 10000
_NP = 10240
_H1 = 16

_NC = 2
_NS = 16
_NW = _NC * _NS
_G = 128
_RPT = _NP // _NS


def _tc_linear1(x8, W1T):
    nr = x8.shape[0]

    def body(x_ref, wt_ref, o_ref):
        wt = wt_ref[...]
        for i in range(8):
            o_ref[pl.ds(0, nr), pl.ds(i * _H1, _H1)] = jax.lax.dot_general(
                x_ref[:, i, :], wt, (((1,), (1,)), ((), ())),
                preferred_element_type=jnp.float32)
        o_ref[pl.ds(nr, _NP // 8 - nr), :] = jnp.zeros(
            (_NP // 8 - nr, 128), jnp.float32)

    return pl.pallas_call(
        body,
        out_shape=jax.ShapeDtypeStruct((_NP // 8, 128), jnp.float32),
    )(x8, W1T)


def _tc_mid(p, b1r, W2p):
    def body(p_ref, b_ref, w_ref, o_ref):
        w = w_ref[...]
        b = b_ref[...]
        for i in range(8):
            sl = pl.ds(i * _H1, _H1)
            h = jnp.maximum(p_ref[0, :, sl] + p_ref[1, :, sl] + b, 0.0)
            o_ref[:, sl] = jnp.dot(h, w, preferred_element_type=jnp.float32)

    return pl.pallas_call(
        body,
        out_shape=jax.ShapeDtypeStruct((p.shape[1], 128), jnp.float32),
    )(p, b1r, W2p)


def _tc_final(p, b2r):
    def body(p_ref, b_ref, o_ref):
        b = b_ref[...]
        for i in range(8):
            sl = pl.ds(i * _H1, _H1)
            o_ref[:, sl] = (p_ref[0, pl.ds(0, _N // 8), sl]
                            + p_ref[1, pl.ds(0, _N // 8), sl] + b)

    return pl.pallas_call(
        body,
        out_shape=jax.ShapeDtypeStruct((_N // 8, 128), jnp.float32),
    )(p, b2r)


def _make_agg(num_groups):
    mesh = plsc.VectorSubcoreMesh(core_axis_name="c", subcore_axis_name="s")
    base = num_groups // _NW
    extra = num_groups % _NW

    @functools.partial(
        pl.kernel,
        out_type=jax.ShapeDtypeStruct((_NC, _NP, _H1), jnp.float32),
        mesh=mesh,
        compiler_params=pltpu.CompilerParams(use_tc_tiling_on_sc=False),
        scratch_types=[
            pltpu.VMEM_SHARED((_NP, _H1), jnp.float32),
            pltpu.VMEM_SHARED((_NP, _H1), jnp.float32),
            pltpu.VMEM((base + 1, 2, _G), jnp.int32),
            pltpu.VMEM((base + 1, _G), jnp.float32),
            pltpu.VMEM((2, 8, _G, _H1), jnp.float32),
            pltpu.VMEM((2, 8, _G, _H1), jnp.float32),
            pltpu.VMEM((_RPT, _H1), jnp.float32),
            pltpu.SemaphoreType.DMA((2, 8)),
            pltpu.SemaphoreType.DMA((2,)),
            pltpu.SemaphoreType.DMA,
            pltpu.SemaphoreType.DMA,
            pltpu.SemaphoreType.DMA,
        ],
    )
    def agg(table_hbm, ei_hbm, ew_hbm, out_hbm,
            table_sh, acc_sh, ei_v, w_v, rows_v, srows_v, zbuf,
            gsem, ssem, esem, tsem, stsem):
        c = lax.axis_index("c")
        s = lax.axis_index("s")
        wid = s * _NC + c
        ng = base + jnp.where(wid < extra, 1, 0)
        g0 = wid * base + jnp.minimum(wid, extra)
        r0 = s * _RPT

        st1 = pltpu.async_copy(table_hbm.at[pl.ds(r0, _RPT)],
                               table_sh.at[pl.ds(r0, _RPT)], stsem)
        e1 = pltpu.async_copy(ei_hbm.at[pl.ds(g0, base)],
                              ei_v.at[pl.ds(0, base)], esem)
        e2 = pltpu.async_copy(ew_hbm.at[pl.ds(g0, base)],
                              w_v.at[pl.ds(0, base)], esem)

        @pl.when(wid < extra)
        def _fetch_tail():
            pltpu.async_copy(ei_hbm.at[pl.ds(g0 + base, 1)],
                             ei_v.at[pl.ds(base, 1)], tsem)
            pltpu.async_copy(ew_hbm.at[pl.ds(g0 + base, 1)],
                             w_v.at[pl.ds(base, 1)], tsem)

        @plsc.parallel_loop(0, _RPT)
        def _zrow(j):
            zbuf[j, :] = jnp.zeros((_H1,), jnp.float32)
        pltpu.sync_copy(zbuf, acc_sh.at[pl.ds(r0, _RPT)])
        st1.wait()
        plsc.subcore_barrier()
        e1.wait()
        e2.wait()

        @pl.when(wid < extra)
        def _wait_tail():
            pltpu.make_async_copy(ei_hbm.at[pl.ds(g0 + base, 1)],
                                  ei_v.at[pl.ds(base, 1)], tsem).wait()
            pltpu.make_async_copy(ew_hbm.at[pl.ds(g0 + base, 1)],
                                  w_v.at[pl.ds(base, 1)], tsem).wait()

        nch = (base + 1 + 7) // 8

        def gather_args(g, p, a):
            return (table_sh.at[ei_v.at[g, 1]], rows_v.at[p, a],
                    gsem.at[p, a])

        def scatter_args(g, p, a):
            return (srows_v.at[p, a], acc_sh.at[ei_v.at[g, 0]], ssem.at[p])

        def issue_chunk_gathers(i, p):
            for a in range(8):
                g = i * 8 + a

                @pl.when(jnp.logical_and(i < nch, g < ng))
                def _issue(g=g, a=a):
                    pltpu.async_copy(*gather_args(g, p, a))

        def drain_chunk_scatters(i, p):
            for a in range(8):
                g = i * 8 + a

                @pl.when(jnp.logical_and(i >= 0, g < ng))
                def _drain(g=g, a=a):
                    pltpu.make_async_copy(*scatter_args(g, p, a)).wait()

        issue_chunk_gathers(jnp.int32(0), 0)
        assert nch % 2 == 0

        def process_chunk(i, p):
            for a in range(8):
                g = i * 8 + a

                @pl.when(g < ng)
                def _process(g=g, a=a):
                    pltpu.make_async_copy(*gather_args(g, p, a)).wait()

                    @plsc.parallel_loop(0, _G // 16)
                    def _scale(b):
                        j0 = b * 16
                        wvec = w_v[g, pl.ds(j0, 16)]
                        for r in range(16):
                            srows_v[p, a, j0 + r, :] = (
                                rows_v[p, a, j0 + r, :] * wvec[r])

                    pltpu.async_copy(*scatter_args(g, p, a), add=True)

        def pair_body(i2, carry):
            for k in range(2):
                i = i2 * 2 + k
                drain_chunk_scatters(i - 2, k)
                issue_chunk_gathers(i + 1, 1 - k)
                process_chunk(i, k)
            return carry

        lax.fori_loop(0, nch // 2, pair_body, 0)
        for i in (nch - 2, nch - 1):
            drain_chunk_scatters(jnp.int32(i), i % 2)
        plsc.subcore_barrier()
        pltpu.sync_copy(acc_sh.at[pl.ds(r0, _RPT)], out_hbm.at[c, pl.ds(r0, _RPT)])

    return agg


def kernel(x, edge_index, edge_weight, W1, b1, W2, b2):
    E = edge_weight.shape[0]
    H2 = W2.shape[1]
    num_groups = E // _G
    assert num_groups * _G == E

    ei_t = edge_index.reshape(2, num_groups, _G).transpose(1, 0, 2)
    ew_r = edge_weight.reshape(num_groups, _G)

    agg = _make_agg(num_groups)

    xw128 = _tc_linear1(x.reshape(_N // 8, 8, 128), W1.T)
    p1 = agg(xw128.reshape(_NP, _H1), ei_t, ew_r)

    W2p = jnp.pad(W2, ((0, 0), (0, _H1 - H2)))
    hw128 = _tc_mid(p1.reshape(_NC, _NP // 8, 128), b1.reshape(1, _H1), W2p)
    p2 = agg(hw128.reshape(_NP, _H1), ei_t, ew_r)

    b2r = jnp.pad(b2, (0, _H1 - H2)).reshape(1, _H1)
    out128 = _tc_final(p2.reshape(_NC, _NP // 8, 128), b2r)
    return out128.reshape(_N, _H1)[:, :H2]
